# baseline (speedup 1.0000x reference)
import functools

import jax
import jax.numpy as jnp
from jax import lax
from jax.experimental import pallas as pl
from jax.experimental.pallas import tpu as pltpu
from jax.experimental.pallas import tpu_sc as plsc

NUM_FIELDS = 26
VOCAB = 100000
EMBED_DIM = 32
BATCH = 16384

NUM_CORES = 2
NUM_SUBCORES = 16
NW = NUM_CORES * NUM_SUBCORES
CHUNK = 4096
NCH = BATCH // CHUNK
LANES = 16
UNROLL = 4


@functools.partial(
    pl.kernel,
    out_type=[
        jax.ShapeDtypeStruct((NUM_FIELDS, BATCH), jnp.float32),
        jax.ShapeDtypeStruct((NUM_FIELDS * EMBED_DIM, BATCH), jnp.float32),
    ],
    mesh=plsc.VectorSubcoreMesh(core_axis_name="c", subcore_axis_name="s"),
    compiler_params=pltpu.CompilerParams(
        use_tc_tiling_on_sc=True, needs_layout_passes=False),
    scratch_types=[
        pltpu.VMEM((1, VOCAB), jnp.float32),
        pltpu.VMEM((2, CHUNK), jnp.int32),
        pltpu.VMEM((2, CHUNK), jnp.float32),
        pltpu.SemaphoreType.DMA,
        pltpu.SemaphoreType.DMA,
        pltpu.SemaphoreType.DMA,
    ],
)
def _lookup(feats_t, tab2, fo2, fo_out, emb_out,
            row_v, idx_v, out_v, sem_i0, sem_i1, sem_w):
    wid = lax.axis_index("s") * NUM_CORES + lax.axis_index("c")
    isem = (sem_i0, sem_i1)
    upper = NUM_FIELDS + jnp.where(wid < NUM_FIELDS, 1, 0)

    def drain_write():
        pltpu.make_async_copy(
            out_v.at[0], emb_out.at[0, pl.ds(0, CHUNK)], sem_w).wait()

    def task_body(t, _):
        fld = jnp.where(t < NUM_FIELDS, t, wid)
        for b in range(2):
            pltpu.async_copy(
                feats_t.at[fld, pl.ds(b * CHUNK, CHUNK)], idx_v.at[b], isem[b])

        @pl.when(t < NUM_FIELDS)
        def _():
            pltpu.sync_copy(tab2.at[t * EMBED_DIM + wid], row_v.at[0])

        @pl.when(t == NUM_FIELDS)
        def _():
            pltpu.sync_copy(fo2.at[wid], row_v.at[0])

        for c in range(NCH):
            b = c % 2
            pltpu.make_async_copy(
                feats_t.at[0, pl.ds(0, CHUNK)], idx_v.at[b], isem[b]).wait()

            @plsc.parallel_loop(0, CHUNK, LANES, unroll=UNROLL)
            def _g(o):
                iv = idx_v[b, pl.ds(o, LANES)]
                out_v[b, pl.ds(o, LANES)] = plsc.load_gather(
                    row_v.at[0], [iv])
            if c + 2 < NCH:
                pltpu.async_copy(
                    feats_t.at[fld, pl.ds((c + 2) * CHUNK, CHUNK)],
                    idx_v.at[b], isem[b])
            if c == 0:
                @pl.when(t != 0)
                def _():
                    drain_write()
            else:
                drain_write()

            @pl.when(t < NUM_FIELDS)
            def _():
                pltpu.async_copy(
                    out_v.at[b],
                    emb_out.at[t * EMBED_DIM + wid, pl.ds(c * CHUNK, CHUNK)],
                    sem_w)

            @pl.when(t == NUM_FIELDS)
            def _():
                pltpu.async_copy(
                    out_v.at[b],
                    fo_out.at[wid, pl.ds(c * CHUNK, CHUNK)], sem_w)
        return 0

    lax.fori_loop(0, upper, task_body, 0)
    drain_write()


def kernel(features_batch, tables, fo_tables):
    feats_t = features_batch.astype(jnp.int32).T
    tab2 = jnp.transpose(tables, (0, 2, 1)).reshape(
        NUM_FIELDS * EMBED_DIM, VOCAB)
    fo2 = fo_tables.reshape(NUM_FIELDS, VOCAB)
    fo_t, emb2 = _lookup(feats_t, tab2, fo2)
    emb = jnp.transpose(
        emb2.reshape(NUM_FIELDS, EMBED_DIM, BATCH), (2, 0, 1))
    fo = fo_t.T[:, :, None]
    return fo, emb


# speedup vs baseline: 1.0112x; 1.0112x over previous
import functools

import jax
import jax.numpy as jnp
from jax import lax
from jax.experimental import pallas as pl
from jax.experimental.pallas import tpu as pltpu
from jax.experimental.pallas import tpu_sc as plsc

NUM_FIELDS = 26
VOCAB = 100000
EMBED_DIM = 32
BATCH = 16384

NUM_CORES = 2
NUM_SUBCORES = 16
NW = NUM_CORES * NUM_SUBCORES
CHUNK = 4096
NCH = BATCH // CHUNK
LANES = 16
UNROLL = 8


@functools.partial(
    pl.kernel,
    out_type=[
        jax.ShapeDtypeStruct((NUM_FIELDS, BATCH), jnp.float32),
        jax.ShapeDtypeStruct((NUM_FIELDS * EMBED_DIM, BATCH), jnp.float32),
    ],
    mesh=plsc.VectorSubcoreMesh(core_axis_name="c", subcore_axis_name="s"),
    compiler_params=pltpu.CompilerParams(
        use_tc_tiling_on_sc=True, needs_layout_passes=False),
    scratch_types=[
        pltpu.VMEM((1, VOCAB), jnp.float32),
        pltpu.VMEM((2, CHUNK), jnp.int32),
        pltpu.VMEM((2, CHUNK), jnp.float32),
        pltpu.SemaphoreType.DMA,
        pltpu.SemaphoreType.DMA,
        pltpu.SemaphoreType.DMA,
    ],
)
def _lookup(feats_t, tab2, fo2, fo_out, emb_out,
            row_v, idx_v, out_v, sem_i0, sem_i1, sem_w):
    wid = lax.axis_index("s") * NUM_CORES + lax.axis_index("c")
    isem = (sem_i0, sem_i1)
    upper = NUM_FIELDS + jnp.where(wid < NUM_FIELDS, 1, 0)

    def drain_write():
        pltpu.make_async_copy(
            out_v.at[0], emb_out.at[0, pl.ds(0, CHUNK)], sem_w).wait()

    def task_body(t, _):
        fld = jnp.where(t < NUM_FIELDS, t, wid)
        for b in range(2):
            pltpu.async_copy(
                feats_t.at[fld, pl.ds(b * CHUNK, CHUNK)], idx_v.at[b], isem[b])

        @pl.when(t < NUM_FIELDS)
        def _():
            pltpu.sync_copy(tab2.at[t * EMBED_DIM + wid], row_v.at[0])

        @pl.when(t == NUM_FIELDS)
        def _():
            pltpu.sync_copy(fo2.at[wid], row_v.at[0])

        for c in range(NCH):
            b = c % 2
            pltpu.make_async_copy(
                feats_t.at[0, pl.ds(0, CHUNK)], idx_v.at[b], isem[b]).wait()

            @plsc.parallel_loop(0, CHUNK, LANES, unroll=UNROLL)
            def _g(o):
                iv = idx_v[b, pl.ds(o, LANES)]
                out_v[b, pl.ds(o, LANES)] = plsc.load_gather(
                    row_v.at[0], [iv])
            if c + 2 < NCH:
                pltpu.async_copy(
                    feats_t.at[fld, pl.ds((c + 2) * CHUNK, CHUNK)],
                    idx_v.at[b], isem[b])
            if c == 0:
                @pl.when(t != 0)
                def _():
                    drain_write()
            else:
                drain_write()

            @pl.when(t < NUM_FIELDS)
            def _():
                pltpu.async_copy(
                    out_v.at[b],
                    emb_out.at[t * EMBED_DIM + wid, pl.ds(c * CHUNK, CHUNK)],
                    sem_w)

            @pl.when(t == NUM_FIELDS)
            def _():
                pltpu.async_copy(
                    out_v.at[b],
                    fo_out.at[wid, pl.ds(c * CHUNK, CHUNK)], sem_w)
        return 0

    lax.fori_loop(0, upper, task_body, 0)
    drain_write()


def kernel(features_batch, tables, fo_tables):
    feats_t = features_batch.astype(jnp.int32).T
    tab2 = jnp.transpose(tables, (0, 2, 1)).reshape(
        NUM_FIELDS * EMBED_DIM, VOCAB)
    fo2 = fo_tables.reshape(NUM_FIELDS, VOCAB)
    fo_t, emb2 = _lookup(feats_t, tab2, fo2)
    emb = jnp.transpose(
        emb2.reshape(NUM_FIELDS, EMBED_DIM, BATCH), (2, 0, 1))
    fo = fo_t.T[:, :, None]
    return fo, emb
